# Initial kernel scaffold; baseline (speedup 1.0000x reference)
import jax
import jax.numpy as jnp
from jax.experimental import pallas as pl


def kernel(hidden_states, W_gate, expert_bias, w1, w2, w1_shared, w2_shared):
    raise NotImplementedError("write your pallas kernel here")


# speedup vs baseline: 2.6511x; 2.6511x over previous
import functools

import jax
import jax.numpy as jnp
from jax.experimental import pallas as pl

T = 2048
H = 1024
E = 8
K = 2
F = 512
FS = 512
ROUTE_SCALE = 1.0

TB = 256


def _router_shared_body(x_ref, wg_ref, bias_ref, w1s_ref, w2s_ref,
                        shared_ref, comb_ref):
    x = x_ref[...]
    gu = jax.lax.dot_general(x, w1s_ref[...], (((1,), (1,)), ((), ())),
                             preferred_element_type=jnp.float32)
    act = jax.nn.silu(gu[:, :FS]) * gu[:, FS:]
    shared_ref[...] = jax.lax.dot_general(
        act, w2s_ref[...], (((1,), (1,)), ((), ())),
        preferred_element_type=jnp.float32)

    logits = jax.lax.dot_general(x, wg_ref[...], (((1,), (1,)), ((), ())),
                                 preferred_element_type=jnp.float32)
    s = jax.nn.sigmoid(logits)
    b = s + bias_ref[...]
    iota = jax.lax.broadcasted_iota(jnp.int32, (TB, E), 1)
    m1 = jnp.max(b, axis=1, keepdims=True)
    id1 = jnp.min(jnp.where(b == m1, iota, E), axis=1, keepdims=True)
    w1v = jnp.sum(jnp.where(iota == id1, s, 0.0), axis=1, keepdims=True)
    b2 = jnp.where(iota == id1, -jnp.inf, b)
    m2 = jnp.max(b2, axis=1, keepdims=True)
    id2 = jnp.min(jnp.where(b2 == m2, iota, E), axis=1, keepdims=True)
    w2v = jnp.sum(jnp.where(iota == id2, s, 0.0), axis=1, keepdims=True)
    denom = jnp.maximum(w1v + w2v, 1e-20)
    comb_ref[...] = (jnp.where(iota == id1, w1v, 0.0)
                     + jnp.where(iota == id2, w2v, 0.0)) / denom


def _experts_body(x_ref, w1_ref, w2_ref, combt_ref, shared_ref, out_ref):
    e = pl.program_id(0)
    x = x_ref[...]
    gu = jax.lax.dot_general(x, w1_ref[0], (((1,), (1,)), ((), ())),
                             preferred_element_type=jnp.float32)
    act = jax.nn.silu(gu[:, :F]) * gu[:, F:]
    ye = jax.lax.dot_general(act, w2_ref[0], (((1,), (1,)), ((), ())),
                             preferred_element_type=jnp.float32)
    contrib = combt_ref[0, 0][:, None] * ye * ROUTE_SCALE

    @pl.when(e == 0)
    def _():
        out_ref[...] = shared_ref[...] + contrib

    @pl.when(e > 0)
    def _():
        out_ref[...] = out_ref[...] + contrib


@jax.jit
def kernel(hidden_states, W_gate, expert_bias, w1, w2, w1_shared, w2_shared):
    x = hidden_states
    bias2d = expert_bias.reshape(1, E)

    shared_out, combine = pl.pallas_call(
        _router_shared_body,
        grid=(T // TB,),
        in_specs=[
            pl.BlockSpec((TB, H), lambda i: (i, 0)),
            pl.BlockSpec((E, H), lambda i: (0, 0)),
            pl.BlockSpec((1, E), lambda i: (0, 0)),
            pl.BlockSpec((2 * FS, H), lambda i: (0, 0)),
            pl.BlockSpec((H, FS), lambda i: (0, 0)),
        ],
        out_specs=[
            pl.BlockSpec((TB, H), lambda i: (i, 0)),
            pl.BlockSpec((TB, E), lambda i: (i, 0)),
        ],
        out_shape=[
            jax.ShapeDtypeStruct((T, H), jnp.float32),
            jax.ShapeDtypeStruct((T, E), jnp.float32),
        ],
    )(x, W_gate, bias2d, w1_shared, w2_shared)

    combt = combine.T.reshape(E, 1, T)

    out = pl.pallas_call(
        _experts_body,
        grid=(E,),
        in_specs=[
            pl.BlockSpec((T, H), lambda e: (0, 0)),
            pl.BlockSpec((1, 2 * F, H), lambda e: (e, 0, 0)),
            pl.BlockSpec((1, H, F), lambda e: (e, 0, 0)),
            pl.BlockSpec((1, 1, T), lambda e: (e, 0, 0)),
            pl.BlockSpec((T, H), lambda e: (0, 0)),
        ],
        out_specs=pl.BlockSpec((T, H), lambda e: (0, 0)),
        out_shape=jax.ShapeDtypeStruct((T, H), jnp.float32),
    )(x, w1, w2, combt, shared_out)

    return out
